# Initial kernel scaffold; baseline (speedup 1.0000x reference)
import jax
import jax.numpy as jnp
from jax.experimental import pallas as pl


def kernel(input_tensor, W_onset, W_medial, W_nucleus, W_coda):
    raise NotImplementedError("write your pallas kernel here")


# speedup vs baseline: 6.6145x; 6.6145x over previous
import functools
import math

import jax
import jax.numpy as jnp
import numpy as np
from jax import lax
from jax.experimental import pallas as pl
from jax.experimental.pallas import tpu as pltpu
from jax.experimental.pallas import tpu_sc as plsc

VOCAB = 100000
D_MODEL = 128
PD = D_MODEL // 4
B, L = 4096, 200
BT = B * L
NW = 32
TOK_W = BT // NW
BLK_TOK = 128
NBLK = TOK_W // BLK_TOK
ROWS_BLK = BLK_TOK * 4
PE_ROWS = L + BLK_TOK
W128_ROWS = VOCAB * PD // 128
GRP_TILE = W128_ROWS // 4
SCHUNK = 50


def _pe_rows() -> np.ndarray:
    position = np.arange(L, dtype=np.float32)[:, None]
    div_term = np.exp(
        np.arange(0, D_MODEL, 2, dtype=np.float32) * (-math.log(10000.0) / D_MODEL)
    )
    pe = np.zeros((L, D_MODEL), dtype=np.float32)
    pe[:, 0::2] = np.sin(position * div_term)
    pe[:, 1::2] = np.cos(position * div_term)
    return np.concatenate([pe, pe[:PE_ROWS - L]], axis=0)


_mesh = plsc.VectorSubcoreMesh(core_axis_name="c", subcore_axis_name="s")


@functools.partial(
    pl.kernel,
    mesh=_mesh,
    out_type=(
        jax.ShapeDtypeStruct((BT, D_MODEL), jnp.float32),
        jax.ShapeDtypeStruct((2, 4 * VOCAB, PD), jnp.float32),
    ),
    scratch_types=[
        pltpu.VMEM((2, 8, 128), jnp.int32),
        pltpu.VMEM((2, ROWS_BLK, PD), jnp.float32),
        pltpu.VMEM((BLK_TOK, D_MODEL), jnp.float32),
        pltpu.VMEM((PE_ROWS, D_MODEL), jnp.float32),
        pltpu.VMEM((SCHUNK, 128), jnp.float32),
        pltpu.VMEM((4 * SCHUNK, PD), jnp.float32),
        pltpu.SemaphoreType.DMA,
        pltpu.SemaphoreType.DMA,
    ],
    compiler_params=pltpu.CompilerParams(use_tc_tiling_on_sc=False),
)
def _sc_embed(w0, w1, w2, w3, idx_hbm, pe_hbm, out_hbm, wstack,
              idx_v, emb_v, out_v, pe_v, st_in, st_out, sem0, sem1):
    cid = lax.axis_index("c")
    sid = lax.axis_index("s")
    wid = sid * 2 + cid
    pltpu.sync_copy(pe_hbm, pe_v)

    for c, wt in enumerate((w0, w1, w2, w3)):
        @pl.when(sid // 4 == c)
        def _stage(c=c, wt=wt):
            j0 = (sid % 4) * GRP_TILE

            def chunk(k, _):
                j = j0 + k * SCHUNK
                pltpu.sync_copy(wt.at[pl.ds(j, SCHUNK)], st_in)

                @plsc.parallel_loop(0, SCHUNK, 1, unroll=4)
                def _(r):
                    for s in range(4):
                        for h in range(2):
                            st_out[4 * r + s, pl.ds(16 * h, 16)] = (
                                st_in[r, pl.ds(32 * s + 16 * h, 16)]
                            )

                pltpu.sync_copy(
                    st_out,
                    wstack.at[cid, pl.ds(4 * (c * W128_ROWS + j), 4 * SCHUNK)],
                )
                return _

            lax.fori_loop(0, GRP_TILE // SCHUNK, chunk, None)

    plsc.subcore_barrier()

    my_w = wstack.at[cid]
    off = lax.rem(lax.iota(jnp.int32, 16), 4) * VOCAB
    idx_row0 = wid * (TOK_W * 4 // 128)
    out_row0 = wid * TOK_W
    sems = (sem0, sem1)

    def fetch(b, islot, eslot, load_idx):
        if load_idx:
            pltpu.sync_copy(
                idx_hbm.at[pl.ds(idx_row0 + (b // 2) * 8, 8)], idx_v.at[islot]
            )
            for g in range(8):
                for m in range(8):
                    idx_v[islot, g, pl.ds(m * 16, 16)] += off
        half = (b % 2) * 4
        for g in range(4):
            pltpu.make_async_copy(
                my_w.at[idx_v.at[islot, half + g]],
                emb_v.at[eslot, pl.ds(g * 128, 128)],
                sems[eslot],
            ).start()

    def process(b, eslot):
        pltpu.make_async_copy(
            my_w.at[pl.ds(0, ROWS_BLK)], emb_v.at[eslot], sems[eslot]
        ).wait()
        pb = lax.rem(b * BLK_TOK, L)

        @plsc.parallel_loop(0, BLK_TOK, 1, unroll=4)
        def _(q):
            pr = pb + q
            for s in range(4):
                for h in range(2):
                    out_v[q, pl.ds(32 * s + 16 * h, 16)] = (
                        emb_v[eslot, 4 * q + s, pl.ds(16 * h, 16)]
                        + pe_v[pr, pl.ds(32 * s + 16 * h, 16)]
                    )

        pltpu.sync_copy(
            out_v, out_hbm.at[pl.ds(out_row0 + b * BLK_TOK, BLK_TOK)]
        )

    fetch(0, 0, 0, True)

    def quad(i, _):
        b0 = i * 4
        fetch(b0 + 1, 0, 1, False)
        process(b0, 0)
        fetch(b0 + 2, 1, 0, True)
        process(b0 + 1, 1)
        fetch(b0 + 3, 1, 1, False)
        process(b0 + 2, 0)

        @pl.when(b0 + 4 < NBLK)
        def _prefetch():
            fetch(b0 + 4, 0, 0, True)

        process(b0 + 3, 1)
        return _

    lax.fori_loop(0, NBLK // 4, quad, None)


def kernel(input_tensor, W_onset, W_medial, W_nucleus, W_coda):
    idx = input_tensor.reshape(BT * 4 // 128, 128)
    pe = jnp.asarray(_pe_rows())
    out, _ = _sc_embed(
        W_onset.reshape(W128_ROWS, 128),
        W_medial.reshape(W128_ROWS, 128),
        W_nucleus.reshape(W128_ROWS, 128),
        W_coda.reshape(W128_ROWS, 128),
        idx,
        pe,
    )
    return out.reshape(B, L, D_MODEL)


# speedup vs baseline: 12.9121x; 1.9521x over previous
import functools
import math

import jax
import jax.numpy as jnp
import numpy as np
from jax import lax
from jax.experimental import pallas as pl
from jax.experimental.pallas import tpu as pltpu
from jax.experimental.pallas import tpu_sc as plsc

VOCAB = 100000
D_MODEL = 128
PD = D_MODEL // 4
B, L = 4096, 200
BT = B * L
NW = 32
NPAIR = L * 4
PAIR_W = NPAIR // NW
NCHUNK = B // 128
CHUNKS_W = PAIR_W * NCHUNK
W128_ROWS = VOCAB * PD // 128
GRP_TILE = W128_ROWS // 4
SCHUNK = 50


def _pe_vals() -> np.ndarray:
    position = np.arange(L, dtype=np.float32)[:, None]
    div_term = np.exp(
        np.arange(0, D_MODEL, 2, dtype=np.float32) * (-math.log(10000.0) / D_MODEL)
    )
    pe = np.zeros((L, D_MODEL), dtype=np.float32)
    pe[:, 0::2] = np.sin(position * div_term)
    pe[:, 1::2] = np.cos(position * div_term)
    return pe


_mesh = plsc.VectorSubcoreMesh(core_axis_name="c", subcore_axis_name="s")


@functools.partial(
    pl.kernel,
    mesh=_mesh,
    out_type=(
        jax.ShapeDtypeStruct((B, L * D_MODEL), jnp.float32),
        jax.ShapeDtypeStruct((2, 4 * VOCAB, PD), jnp.float32),
    ),
    scratch_types=[
        pltpu.VMEM((NCHUNK, 128), jnp.int32),
        pltpu.VMEM((2, 128), jnp.int32),
        pltpu.VMEM((2, 128, PD), jnp.float32),
        pltpu.VMEM((L, D_MODEL), jnp.float32),
        pltpu.VMEM((SCHUNK, 128), jnp.float32),
        pltpu.VMEM((4 * SCHUNK, PD), jnp.float32),
        pltpu.SemaphoreType.DMA,
        pltpu.SemaphoreType.DMA,
    ],
    compiler_params=pltpu.CompilerParams(use_tc_tiling_on_sc=False),
)
def _sc_embed(w0, w1, w2, w3, idx_hbm, pe_hbm, out_hbm, wstack,
              idx_v, row_v, emb_v, pe_v, st_in, st_out, sem0, sem1):
    cid = lax.axis_index("c")
    sid = lax.axis_index("s")
    wid = sid * 2 + cid
    pltpu.sync_copy(pe_hbm, pe_v)

    for c, wt in enumerate((w0, w1, w2, w3)):
        @pl.when(sid // 4 == c)
        def _stage(c=c, wt=wt):
            j0 = (sid % 4) * GRP_TILE

            def chunk(k, _):
                j = j0 + k * SCHUNK
                pltpu.sync_copy(wt.at[pl.ds(j, SCHUNK)], st_in)

                @plsc.parallel_loop(0, SCHUNK, 1, unroll=4)
                def _(r):
                    for s in range(4):
                        for h in range(2):
                            st_out[4 * r + s, pl.ds(16 * h, 16)] = (
                                st_in[r, pl.ds(32 * s + 16 * h, 16)]
                            )

                pltpu.sync_copy(
                    st_out,
                    wstack.at[cid, pl.ds(4 * (c * W128_ROWS + j), 4 * SCHUNK)],
                )
                return _

            lax.fori_loop(0, GRP_TILE // SCHUNK, chunk, None)

    plsc.subcore_barrier()

    my_w = wstack.at[cid]
    pair0 = wid * PAIR_W
    sems = (sem0, sem1)

    def fetch(t, eslot):
        pl_loc = t // NCHUNK
        j = lax.rem(t, NCHUNK)
        pair = pair0 + pl_loc
        c = lax.rem(pair, 4)

        @pl.when(j == 0)
        def _load_pair():
            pltpu.sync_copy(idx_hbm.at[pl.ds(pair * NCHUNK, NCHUNK)], idx_v)

        off = jnp.full((16,), 0, jnp.int32) + c * VOCAB
        for m in range(8):
            row_v[eslot, pl.ds(16 * m, 16)] = idx_v[j, pl.ds(16 * m, 16)] + off
        pltpu.make_async_copy(
            my_w.at[row_v.at[eslot]], emb_v.at[eslot], sems[eslot]
        ).start()

    def process(t, eslot):
        pltpu.make_async_copy(
            my_w.at[pl.ds(0, 128)], emb_v.at[eslot], sems[eslot]
        ).wait()
        pair = pair0 + t // NCHUNK
        j = lax.rem(t, NCHUNK)
        l = pair // 4
        c = lax.rem(pair, 4)
        col = l * D_MODEL + c * PD
        pe0 = pe_v[l, pl.ds(c * PD, 16)]
        pe1 = pe_v[l, pl.ds(c * PD + 16, 16)]

        @plsc.parallel_loop(0, 128, 1, unroll=8)
        def _(r):
            plsc.addupdate(emb_v.at[eslot, r, pl.ds(0, 16)], pe0)
            plsc.addupdate(emb_v.at[eslot, r, pl.ds(16, 16)], pe1)

        pltpu.sync_copy(
            emb_v.at[eslot],
            out_hbm.at[pl.ds(j * 128, 128), pl.ds(col, PD)],
        )

    fetch(0, 0)

    def duo(m, _):
        t0 = m * 2
        fetch(t0 + 1, 1)
        process(t0, 0)

        @pl.when(t0 + 2 < CHUNKS_W)
        def _prefetch():
            fetch(t0 + 2, 0)

        process(t0 + 1, 1)
        return _

    lax.fori_loop(0, CHUNKS_W // 2, duo, None)


def kernel(input_tensor, W_onset, W_medial, W_nucleus, W_coda):
    idx = jnp.transpose(input_tensor, (1, 2, 0)).reshape(NPAIR * NCHUNK, 128)
    pe = jnp.asarray(_pe_vals())
    out, _ = _sc_embed(
        W_onset.reshape(W128_ROWS, 128),
        W_medial.reshape(W128_ROWS, 128),
        W_nucleus.reshape(W128_ROWS, 128),
        W_coda.reshape(W128_ROWS, 128),
        idx,
        pe,
    )
    return out.reshape(B, L, D_MODEL)


# speedup vs baseline: 17.1897x; 1.3313x over previous
import functools
import math

import jax
import jax.numpy as jnp
import numpy as np
from jax import lax
from jax.experimental import pallas as pl
from jax.experimental.pallas import tpu as pltpu
from jax.experimental.pallas import tpu_sc as plsc

VOCAB = 100000
D_MODEL = 128
PD = D_MODEL // 4
B, L = 4096, 200
NW = 32
NPAIR = L * 4
PAIR_W = NPAIR // NW
IDXR = B // 128
CHUNK_B = 256
NCH_P = B // CHUNK_B
CHUNKS_W = PAIR_W * NCH_P
NSLOT = 4
W128_ROWS = VOCAB * PD // 128
GRP_TILE = W128_ROWS // 4
SCHUNK = 25
NSCH = GRP_TILE // SCHUNK


def _pe_vals() -> np.ndarray:
    position = np.arange(L, dtype=np.float32)[:, None]
    div_term = np.exp(
        np.arange(0, D_MODEL, 2, dtype=np.float32) * (-math.log(10000.0) / D_MODEL)
    )
    pe = np.zeros((L, D_MODEL), dtype=np.float32)
    pe[:, 0::2] = np.sin(position * div_term)
    pe[:, 1::2] = np.cos(position * div_term)
    return pe


_mesh = plsc.VectorSubcoreMesh(core_axis_name="c", subcore_axis_name="s")


@functools.partial(
    pl.kernel,
    mesh=_mesh,
    out_type=(
        jax.ShapeDtypeStruct((B, L * D_MODEL), jnp.float32),
        jax.ShapeDtypeStruct((2, 4 * VOCAB, PD), jnp.float32),
    ),
    scratch_types=[
        pltpu.VMEM((IDXR, 128), jnp.int32),
        pltpu.VMEM((NSLOT, 2, 128), jnp.int32),
        pltpu.VMEM((NSLOT, CHUNK_B, PD), jnp.float32),
        pltpu.VMEM((L, D_MODEL), jnp.float32),
        pltpu.VMEM((2, SCHUNK, 128), jnp.float32),
        pltpu.VMEM((2, 4 * SCHUNK, PD), jnp.float32),
        pltpu.SemaphoreType.DMA,
        pltpu.SemaphoreType.DMA,
        pltpu.SemaphoreType.DMA,
        pltpu.SemaphoreType.DMA,
        pltpu.SemaphoreType.DMA,
        pltpu.SemaphoreType.DMA,
        pltpu.SemaphoreType.DMA,
        pltpu.SemaphoreType.DMA,
    ],
    compiler_params=pltpu.CompilerParams(use_tc_tiling_on_sc=False),
)
def _sc_embed(w0, w1, w2, w3, idx_hbm, pe_hbm, out_hbm, wstack,
              idx_v, row_v, emb_v, pe_v, st_in, st_out,
              g0, g1, g2, g3, si0, si1, so0, so1):
    cid = lax.axis_index("c")
    sid = lax.axis_index("s")
    wid = sid * 2 + cid
    pltpu.sync_copy(pe_hbm, pe_v)

    sisem = (si0, si1)
    sosem = (so0, so1)
    for c, wt in enumerate((w0, w1, w2, w3)):
        @pl.when(sid // 4 == c)
        def _stage(c=c, wt=wt):
            j0 = (sid % 4) * GRP_TILE

            def in_dma(k, buf):
                return pltpu.make_async_copy(
                    wt.at[pl.ds(j0 + k * SCHUNK, SCHUNK)],
                    st_in.at[buf], sisem[buf],
                )

            def out_dma(k, buf):
                return pltpu.make_async_copy(
                    st_out.at[buf],
                    wstack.at[cid, pl.ds(4 * (c * W128_ROWS + j0 + k * SCHUNK),
                                         4 * SCHUNK)],
                    sosem[buf],
                )

            def repack(buf):
                @plsc.parallel_loop(0, SCHUNK, 1, unroll=5)
                def _(r):
                    for s in range(4):
                        for h in range(2):
                            st_out[buf, 4 * r + s, pl.ds(16 * h, 16)] = (
                                st_in[buf, r, pl.ds(32 * s + 16 * h, 16)]
                            )

            in_dma(0, 0).start()

            def stpair(q, _):
                k = q * 2
                for buf in (0, 1):
                    in_dma(k + buf, buf).wait()

                    @pl.when(k + buf + 1 < NSCH)
                    def _nx():
                        in_dma(k + buf + 1, 1 - buf).start()

                    @pl.when(q > 0)
                    def _dr():
                        out_dma(0, buf).wait()

                    repack(buf)
                    out_dma(k + buf, buf).start()
                return _

            lax.fori_loop(0, NSCH // 2, stpair, None)
            out_dma(0, 0).wait()
            out_dma(0, 1).wait()

    plsc.subcore_barrier()

    my_w = wstack.at[cid]
    pair0 = wid * PAIR_W
    gsem = (g0, g1, g2, g3)

    def fetch(t, slot):
        j = lax.rem(t, NCH_P)
        pair = pair0 + t // NCH_P
        c = lax.rem(pair, 4)

        @pl.when(j == 0)
        def _load_pair():
            pltpu.sync_copy(idx_hbm.at[pl.ds(pair * IDXR, IDXR)], idx_v)

        off = jnp.full((16,), 0, jnp.int32) + c * VOCAB
        for h in range(2):
            for m in range(8):
                row_v[slot, h, pl.ds(16 * m, 16)] = (
                    idx_v[j * 2 + h, pl.ds(16 * m, 16)] + off
                )
            pltpu.make_async_copy(
                my_w.at[row_v.at[slot, h]],
                emb_v.at[slot, pl.ds(h * 128, 128)],
                gsem[slot],
            ).start()

    def process(t, slot):
        pltpu.make_async_copy(
            my_w.at[pl.ds(0, CHUNK_B)], emb_v.at[slot], gsem[slot]
        ).wait()
        pair = pair0 + t // NCH_P
        j = lax.rem(t, NCH_P)
        l = pair // 4
        c = lax.rem(pair, 4)
        pe0 = pe_v[l, pl.ds(c * PD, 16)]
        pe1 = pe_v[l, pl.ds(c * PD + 16, 16)]

        @plsc.parallel_loop(0, CHUNK_B, 1, unroll=8)
        def _(r):
            plsc.addupdate(emb_v.at[slot, r, pl.ds(0, 16)], pe0)
            plsc.addupdate(emb_v.at[slot, r, pl.ds(16, 16)], pe1)

        pltpu.sync_copy(
            emb_v.at[slot],
            out_hbm.at[pl.ds(j * CHUNK_B, CHUNK_B),
                       pl.ds(l * D_MODEL + c * PD, PD)],
        )

    fetch(0, 0)
    fetch(1, 1)
    fetch(2, 2)

    def quad(i, _):
        t0 = i * 4
        for u in range(4):
            t = t0 + u

            @pl.when(t + 3 < CHUNKS_W)
            def _pf(t=t, u=u):
                fetch(t + 3, (u + 3) % 4)

            process(t, u)
        return _

    lax.fori_loop(0, CHUNKS_W // 4, quad, None)


def kernel(input_tensor, W_onset, W_medial, W_nucleus, W_coda):
    idx = jnp.transpose(input_tensor, (1, 2, 0)).reshape(NPAIR * IDXR, 128)
    pe = jnp.asarray(_pe_vals())
    out, _ = _sc_embed(
        W_onset.reshape(W128_ROWS, 128),
        W_medial.reshape(W128_ROWS, 128),
        W_nucleus.reshape(W128_ROWS, 128),
        W_coda.reshape(W128_ROWS, 128),
        idx,
        pe,
    )
    return out.reshape(B, L, D_MODEL)


# speedup vs baseline: 17.4209x; 1.0134x over previous
import functools
import math

import jax
import jax.numpy as jnp
import numpy as np
from jax import lax
from jax.experimental import pallas as pl
from jax.experimental.pallas import tpu as pltpu
from jax.experimental.pallas import tpu_sc as plsc

VOCAB = 100000
D_MODEL = 128
PD = D_MODEL // 4
B, L = 4096, 200
NW = 32
NPAIR = L * 4
PAIR_W = NPAIR // NW
IDXR = B // 128
CHUNK_B = 512
ROWS_CH = CHUNK_B // 128
NCH_P = B // CHUNK_B
CHUNKS_W = PAIR_W * NCH_P
NSLOT = 4
W128_ROWS = VOCAB * PD // 128
GRP_TILE = W128_ROWS // 4
SCHUNK = 25
NSCH = GRP_TILE // SCHUNK


def _pe_vals() -> np.ndarray:
    position = np.arange(L, dtype=np.float32)[:, None]
    div_term = np.exp(
        np.arange(0, D_MODEL, 2, dtype=np.float32) * (-math.log(10000.0) / D_MODEL)
    )
    pe = np.zeros((L, D_MODEL), dtype=np.float32)
    pe[:, 0::2] = np.sin(position * div_term)
    pe[:, 1::2] = np.cos(position * div_term)
    return pe


_mesh = plsc.VectorSubcoreMesh(core_axis_name="c", subcore_axis_name="s")


@functools.partial(
    pl.kernel,
    mesh=_mesh,
    out_type=(
        jax.ShapeDtypeStruct((B, L * D_MODEL), jnp.float32),
        jax.ShapeDtypeStruct((2, 4 * VOCAB, PD), jnp.float32),
    ),
    scratch_types=[
        pltpu.VMEM((IDXR, 128), jnp.int32),
        pltpu.VMEM((NSLOT, ROWS_CH, 128), jnp.int32),
        pltpu.VMEM((NSLOT, CHUNK_B, PD), jnp.float32),
        pltpu.VMEM((L, D_MODEL), jnp.float32),
        pltpu.VMEM((2, SCHUNK, 128), jnp.float32),
        pltpu.VMEM((2, 4 * SCHUNK, PD), jnp.float32),
        pltpu.SemaphoreType.DMA,
        pltpu.SemaphoreType.DMA,
        pltpu.SemaphoreType.DMA,
        pltpu.SemaphoreType.DMA,
        pltpu.SemaphoreType.DMA,
        pltpu.SemaphoreType.DMA,
        pltpu.SemaphoreType.DMA,
        pltpu.SemaphoreType.DMA,
        pltpu.SemaphoreType.DMA,
        pltpu.SemaphoreType.DMA,
        pltpu.SemaphoreType.DMA,
        pltpu.SemaphoreType.DMA,
    ],
    compiler_params=pltpu.CompilerParams(use_tc_tiling_on_sc=False),
)
def _sc_embed(w0, w1, w2, w3, idx_hbm, pe_hbm, out_hbm, wstack,
              idx_v, row_v, emb_v, pe_v, st_in, st_out,
              g0, g1, g2, g3, si0, si1, so0, so1, t0s, t1s, t2s, t3s):
    cid = lax.axis_index("c")
    sid = lax.axis_index("s")
    wid = sid * 2 + cid
    pltpu.sync_copy(pe_hbm, pe_v)

    sisem = (si0, si1)
    sosem = (so0, so1)
    for c, wt in enumerate((w0, w1, w2, w3)):
        @pl.when(sid // 4 == c)
        def _stage(c=c, wt=wt):
            j0 = (sid % 4) * GRP_TILE

            def in_dma(k, buf):
                return pltpu.make_async_copy(
                    wt.at[pl.ds(j0 + k * SCHUNK, SCHUNK)],
                    st_in.at[buf], sisem[buf],
                )

            def out_dma(k, buf):
                return pltpu.make_async_copy(
                    st_out.at[buf],
                    wstack.at[cid, pl.ds(4 * (c * W128_ROWS + j0 + k * SCHUNK),
                                         4 * SCHUNK)],
                    sosem[buf],
                )

            def repack(buf):
                @plsc.parallel_loop(0, SCHUNK, 1, unroll=5)
                def _(r):
                    for s in range(4):
                        for h in range(2):
                            st_out[buf, 4 * r + s, pl.ds(16 * h, 16)] = (
                                st_in[buf, r, pl.ds(32 * s + 16 * h, 16)]
                            )

            in_dma(0, 0).start()

            def stpair(q, _):
                k = q * 2
                for buf in (0, 1):
                    in_dma(k + buf, buf).wait()

                    @pl.when(k + buf + 1 < NSCH)
                    def _nx():
                        in_dma(k + buf + 1, 1 - buf).start()

                    @pl.when(q > 0)
                    def _dr():
                        out_dma(0, buf).wait()

                    repack(buf)
                    out_dma(k + buf, buf).start()
                return _

            lax.fori_loop(0, NSCH // 2, stpair, None)
            out_dma(0, 0).wait()
            out_dma(0, 1).wait()

    plsc.subcore_barrier()

    my_w = wstack.at[cid]
    pair0 = wid * PAIR_W
    gsem = (g0, g1, g2, g3)
    ssem = (t0s, t1s, t2s, t3s)

    def store_dma(t, slot):
        pair = pair0 + t // NCH_P
        j = lax.rem(t, NCH_P)
        l = pair // 4
        c = lax.rem(pair, 4)
        return pltpu.make_async_copy(
            emb_v.at[slot],
            out_hbm.at[pl.ds(j * CHUNK_B, CHUNK_B),
                       pl.ds(l * D_MODEL + c * PD, PD)],
            ssem[slot],
        )

    def fetch(t, slot):
        j = lax.rem(t, NCH_P)
        pair = pair0 + t // NCH_P
        c = lax.rem(pair, 4)

        @pl.when(t >= NSLOT)
        def _drain_store():
            store_dma(t - NSLOT, slot).wait()

        @pl.when(j == 0)
        def _load_pair():
            pltpu.sync_copy(idx_hbm.at[pl.ds(pair * IDXR, IDXR)], idx_v)

        off = jnp.full((16,), 0, jnp.int32) + c * VOCAB
        for h in range(ROWS_CH):
            for m in range(8):
                row_v[slot, h, pl.ds(16 * m, 16)] = (
                    idx_v[j * ROWS_CH + h, pl.ds(16 * m, 16)] + off
                )
            pltpu.make_async_copy(
                my_w.at[row_v.at[slot, h]],
                emb_v.at[slot, pl.ds(h * 128, 128)],
                gsem[slot],
            ).start()

    def process(t, slot):
        pltpu.make_async_copy(
            my_w.at[pl.ds(0, CHUNK_B)], emb_v.at[slot], gsem[slot]
        ).wait()
        pair = pair0 + t // NCH_P
        j = lax.rem(t, NCH_P)
        l = pair // 4
        c = lax.rem(pair, 4)
        pe0 = pe_v[l, pl.ds(c * PD, 16)]
        pe1 = pe_v[l, pl.ds(c * PD + 16, 16)]

        @plsc.parallel_loop(0, CHUNK_B, 1, unroll=8)
        def _(r):
            plsc.addupdate(emb_v.at[slot, r, pl.ds(0, 16)], pe0)
            plsc.addupdate(emb_v.at[slot, r, pl.ds(16, 16)], pe1)

        store_dma(t, slot).start()

    fetch(0, 0)
    fetch(1, 1)
    fetch(2, 2)

    def quad(i, _):
        t0 = i * 4
        for u in range(4):
            t = t0 + u

            @pl.when(t + 3 < CHUNKS_W)
            def _pf(t=t, u=u):
                fetch(t + 3, (u + 3) % 4)

            process(t, u)
        return _

    lax.fori_loop(0, CHUNKS_W // 4, quad, None)
    for slot in range(NSLOT):
        store_dma(CHUNKS_W - NSLOT + slot, slot).wait()


def kernel(input_tensor, W_onset, W_medial, W_nucleus, W_coda):
    idx = jnp.transpose(input_tensor, (1, 2, 0)).reshape(NPAIR * IDXR, 128)
    pe = jnp.asarray(_pe_vals())
    out, _ = _sc_embed(
        W_onset.reshape(W128_ROWS, 128),
        W_medial.reshape(W128_ROWS, 128),
        W_nucleus.reshape(W128_ROWS, 128),
        W_coda.reshape(W128_ROWS, 128),
        idx,
        pe,
    )
    return out.reshape(B, L, D_MODEL)
